# baseline (speedup 1.0000x reference)
import functools

import jax
import jax.numpy as jnp
from jax import lax
from jax.experimental import pallas as pl
from jax.experimental.pallas import tpu as pltpu
from jax.experimental.pallas import tpu_sc as plsc

HIDDEN = 64
DCHUNK = 8
NBUF = 2
NSTAGE = HIDDEN // DCHUNK


def _make_sc_kernel(batch):
    info = plsc.get_sparse_core_info()
    nc, ns = info.num_cores, info.num_subcores
    nw = nc * ns
    assert batch % (8 * nw) == 0
    bpw = batch // nw

    mesh = plsc.VectorSubcoreMesh(core_axis_name="c", subcore_axis_name="s")

    @functools.partial(
        pl.kernel,
        mesh=mesh,
        compiler_params=pltpu.CompilerParams(
            needs_layout_passes=False, use_tc_tiling_on_sc=False
        ),
        out_type=jax.ShapeDtypeStruct((batch,), jnp.float32),
        scratch_types=[
            pltpu.VMEM((bpw,), jnp.int32),
            pltpu.VMEM((bpw,), jnp.int32),
            pltpu.VMEM((bpw,), jnp.int32),
            pltpu.VMEM((NBUF, DCHUNK, bpw), jnp.float32),
            pltpu.VMEM((NBUF, DCHUNK, bpw), jnp.float32),
            pltpu.VMEM((NBUF, DCHUNK, bpw), jnp.float32),
            pltpu.VMEM((bpw,), jnp.float32),
            pltpu.SemaphoreType.DMA((NBUF,)),
        ],
    )
    def k(entT_hbm, relT_hbm, hidx_hbm, ridx_hbm, tidx_hbm, out_hbm,
          hidx_v, ridx_v, tidx_v, hbuf, rbuf, tbuf, score_v, sems):
        wid = lax.axis_index("s") * nc + lax.axis_index("c")
        base = wid * bpw
        pltpu.sync_copy(hidx_hbm.at[pl.ds(base, bpw)], hidx_v)
        pltpu.sync_copy(ridx_hbm.at[pl.ds(base, bpw)], ridx_v)
        pltpu.sync_copy(tidx_hbm.at[pl.ds(base, bpw)], tidx_v)

        def fire(stage, slot):
            for j in range(DCHUNK):
                d = stage * DCHUNK + j
                pltpu.async_copy(
                    entT_hbm.at[d].at[hidx_v], hbuf.at[slot, j], sems.at[slot]
                )
                pltpu.async_copy(
                    relT_hbm.at[d].at[ridx_v], rbuf.at[slot, j], sems.at[slot]
                )
                pltpu.async_copy(
                    entT_hbm.at[d].at[tidx_v], tbuf.at[slot, j], sems.at[slot]
                )

        def drain(slot):
            for j in range(DCHUNK):
                pltpu.make_async_copy(
                    entT_hbm.at[0].at[hidx_v], hbuf.at[slot, j], sems.at[slot]
                ).wait()
                pltpu.make_async_copy(
                    relT_hbm.at[0].at[ridx_v], rbuf.at[slot, j], sems.at[slot]
                ).wait()
                pltpu.make_async_copy(
                    entT_hbm.at[0].at[tidx_v], tbuf.at[slot, j], sems.at[slot]
                ).wait()

        def compute(slot):
            def vbody(v, carry):
                sl = pl.ds(v * 16, 16)
                acc = score_v[sl]
                for j in range(DCHUNK):
                    acc = acc + (hbuf[slot, j, sl]
                                 * rbuf[slot, j, sl]
                                 * tbuf[slot, j, sl])
                score_v[sl] = acc
                return carry

            lax.fori_loop(0, bpw // 16, vbody, 0)

        def zbody(v, carry):
            score_v[pl.ds(v * 16, 16)] = jnp.zeros((16,), jnp.float32)
            return carry

        lax.fori_loop(0, bpw // 16, zbody, 0)

        fire(0, 0)

        def stage_pair(p, carry):
            s0 = p * 2
            fire(s0 + 1, 1)
            drain(0)
            compute(0)

            @pl.when(s0 + 2 < NSTAGE)
            def _():
                fire(s0 + 2, 0)

            drain(1)
            compute(1)
            return carry

        lax.fori_loop(0, NSTAGE // 2, stage_pair, 0)

        pltpu.sync_copy(score_v, out_hbm.at[pl.ds(base, bpw)])

    return k


@jax.jit
def kernel(entity_embedding, relation_embedding, sample):
    batch = sample.shape[0]
    hidx = sample[:, 0]
    ridx = sample[:, 1]
    tidx = sample[:, 2]
    k = _make_sc_kernel(batch)
    score = k(entity_embedding.T, relation_embedding.T, hidx, ridx, tidx)
    return score.reshape(batch, 1)
